# baseline (speedup 1.0000x reference)
import functools
import math

import jax
import jax.numpy as jnp
from jax import lax
from jax.experimental import pallas as pl
from jax.experimental.pallas import tpu as pltpu
from jax.experimental.pallas import tpu_sc as plsc

AVG_NUM_NEIGHBORS = 64.0
FACTOR = 1.0 / math.sqrt(AVG_NUM_NEIGHBORS)

NC = 2
NS = 16
NW = NC * NS
LANES = 16
ROW = 128
ROWS = 40
CHUNK = ROW * ROWS
NBUF = 3


def _sc_scatter_kernel(E, N):
    assert E % ROW == 0
    e_rows = E // ROW
    assert e_rows % ROWS == 0
    n_chunks = e_rows // ROWS
    cpw = -(-n_chunks // NW)
    cpw += (-cpw) % NBUF
    nps = -(-N // (NS * 8)) * 8
    n_pad = nps * NS

    mesh = plsc.VectorSubcoreMesh(core_axis_name="c", subcore_axis_name="s")

    buf_types = {}
    for b in range(NBUF):
        buf_types[f"idx{b}"] = pltpu.VMEM((ROWS, ROW), jnp.int32)
        buf_types[f"j{b}"] = pltpu.VMEM((CHUNK,), jnp.float32)
        buf_types[f"s{b}"] = pltpu.VMEM((CHUNK,), jnp.float32)
        buf_types[f"eng{b}"] = pltpu.VMEM((CHUNK,), jnp.float32)

    @functools.partial(
        pl.kernel,
        out_type=(
            jax.ShapeDtypeStruct((E,), jnp.float32),
            jax.ShapeDtypeStruct((NC * n_pad,), jnp.float32),
        ),
        mesh=mesh,
        scratch_types=dict(
            **buf_types,
            stage_v=pltpu.VMEM((nps,), jnp.float32),
            acc_sh=pltpu.VMEM_SHARED((n_pad,), jnp.float32),
            in_sem=pltpu.SemaphoreType.DMA,
            scat_sem=pltpu.SemaphoreType.DMA,
            wb_sem=pltpu.SemaphoreType.DMA,
        ),
    )
    def body(center_hbm, j_hbm, s_hbm, eng_hbm, partial_hbm, **refs):
        bufs = [(refs[f"idx{b}"], refs[f"j{b}"], refs[f"s{b}"],
                 refs[f"eng{b}"]) for b in range(NBUF)]
        stage_v = refs["stage_v"]
        acc_sh = refs["acc_sh"]
        in_sem = refs["in_sem"]
        scat_sem = refs["scat_sem"]
        wb_sem = refs["wb_sem"]

        cid = lax.axis_index("c")
        sid = lax.axis_index("s")
        wid = sid * NC + cid

        def zero_body(i, _):
            stage_v[pl.ds(i * LANES, LANES)] = jnp.zeros((LANES,), jnp.float32)
            return 0
        lax.fori_loop(0, nps // LANES, zero_body, 0, unroll=8)
        pltpu.sync_copy(stage_v, acc_sh.at[pl.ds(sid * nps, nps)])
        plsc.subcore_barrier()

        def fire_inputs(c, buf):
            idx_v, j_v, s_v, _ = buf
            pltpu.async_copy(center_hbm.at[pl.ds(c * ROWS, ROWS)], idx_v,
                             in_sem)
            pltpu.async_copy(j_hbm.at[pl.ds(c * CHUNK, CHUNK)], j_v, in_sem)
            pltpu.async_copy(s_hbm.at[pl.ds(c * CHUNK, CHUNK)], s_v, in_sem)

        def wait_inputs(c, buf):
            idx_v, j_v, s_v, _ = buf
            pltpu.make_async_copy(center_hbm.at[pl.ds(c * ROWS, ROWS)],
                                  idx_v, in_sem).wait()
            pltpu.make_async_copy(j_hbm.at[pl.ds(c * CHUNK, CHUNK)], j_v,
                                  in_sem).wait()
            pltpu.make_async_copy(s_hbm.at[pl.ds(c * CHUNK, CHUNK)], s_v,
                                  in_sem).wait()

        def drain_chunk(c, buf):
            idx_v, _, _, eng_v = buf

            def drain_body(r, _):
                pltpu.make_async_copy(eng_v.at[pl.ds(r * ROW, ROW)],
                                      acc_sh.at[idx_v.at[r]],
                                      scat_sem).wait()
                return 0
            lax.fori_loop(0, ROWS, drain_body, 0)
            pltpu.make_async_copy(eng_v, eng_hbm.at[pl.ds(c * CHUNK, CHUNK)],
                                  wb_sem).wait()

        def process(k, i_cur, i_nxt, i_prv):
            c = wid + k * NW
            cp = wid + (k - 2) * NW
            cn = wid + (k + 1) * NW

            @pl.when(c < n_chunks)
            def _():
                wait_inputs(c, bufs[i_cur])

            @pl.when((k >= 2) & (cp < n_chunks))
            def _():
                drain_chunk(cp, bufs[i_prv])

            @pl.when(cn < n_chunks)
            def _():
                fire_inputs(cn, bufs[i_nxt])

            @pl.when(c < n_chunks)
            def _():
                idx_v, j_v, s_v, eng_v = bufs[i_cur]

                def row_body(r, _):
                    for t in range(ROW // LANES):
                        sl = pl.ds(r * ROW + t * LANES, LANES)
                        eng_v[sl] = j_v[sl] * s_v[sl]
                    pltpu.async_copy(eng_v.at[pl.ds(r * ROW, ROW)],
                                     acc_sh.at[idx_v.at[r]], scat_sem,
                                     add=True)
                    return 0
                lax.fori_loop(0, ROWS, row_body, 0)

                pltpu.async_copy(eng_v, eng_hbm.at[pl.ds(c * CHUNK, CHUNK)],
                                 wb_sem)

        fire_inputs(wid, bufs[0])

        def rot_body(p, _):
            for j in range(NBUF):
                k = NBUF * p + j
                process(k, j, (j + 1) % NBUF, (j + 1) % NBUF)
            return 0
        lax.fori_loop(0, cpw // NBUF, rot_body, 0)

        for k in (cpw - 2, cpw - 1):
            c_last = wid + k * NW

            @pl.when(c_last < n_chunks)
            def _(c_last=c_last, k=k):
                drain_chunk(c_last, bufs[k % NBUF])

        plsc.subcore_barrier()
        pltpu.sync_copy(acc_sh.at[pl.ds(sid * nps, nps)], stage_v)
        pltpu.sync_copy(stage_v,
                        partial_hbm.at[pl.ds(cid * n_pad + sid * nps, nps)])

    return body, n_pad


def _combine_kernel(p_ref, o_ref):
    o_ref[...] = (p_ref[0] + p_ref[1]) * FACTOR


def kernel(edge_index, atom_type, edge_J, edge_spin_distance):
    N = atom_type.shape[0]
    E = edge_J.shape[0]
    center2d = edge_index[0].reshape(E // ROW, ROW)
    j_flat = edge_J.reshape(E)

    sc_fn, n_pad = _sc_scatter_kernel(E, N)
    eng_flat, partial = sc_fn(center2d, j_flat, edge_spin_distance)

    p3 = partial.reshape(NC, n_pad // 128, 128)
    atom_pad = pl.pallas_call(
        _combine_kernel,
        out_shape=jax.ShapeDtypeStruct((n_pad // 128, 128), jnp.float32),
    )(p3)
    atom_eng = atom_pad.reshape(n_pad)[:N].reshape(N, 1)
    return eng_flat.reshape(E, 1), atom_eng


# speedup vs baseline: 1.0334x; 1.0334x over previous
import functools
import math

import jax
import jax.numpy as jnp
from jax import lax
from jax.experimental import pallas as pl
from jax.experimental.pallas import tpu as pltpu
from jax.experimental.pallas import tpu_sc as plsc

AVG_NUM_NEIGHBORS = 64.0
FACTOR = 1.0 / math.sqrt(AVG_NUM_NEIGHBORS)

NC = 2
NS = 16
NW = NC * NS
LANES = 16
ROW = 128
ROWS = 80
CHUNK = ROW * ROWS


def _sc_scatter_kernel(E, N):
    assert E % (ROW * ROWS) == 0
    n_chunks = E // CHUNK
    cpw = -(-n_chunks // NW)
    cpw += (-cpw) % 6
    nps = -(-N // (NS * 8)) * 8
    n_pad = nps * NS

    mesh = plsc.VectorSubcoreMesh(core_axis_name="c", subcore_axis_name="s")

    scratch = dict(
        acc_sh=pltpu.VMEM_SHARED((n_pad,), jnp.float32),
        in_sem=pltpu.SemaphoreType.DMA,
        scat_sem=pltpu.SemaphoreType.DMA,
        wb_sem=pltpu.SemaphoreType.DMA,
    )
    for b in range(3):
        scratch[f"idx{b}"] = pltpu.VMEM((ROWS, ROW), jnp.int32)
        scratch[f"eng{b}"] = pltpu.VMEM((CHUNK,), jnp.float32)
    for b in range(2):
        scratch[f"j{b}"] = pltpu.VMEM((CHUNK,), jnp.float32)
        scratch[f"s{b}"] = pltpu.VMEM((CHUNK,), jnp.float32)

    @functools.partial(
        pl.kernel,
        out_type=(
            jax.ShapeDtypeStruct((E,), jnp.float32),
            jax.ShapeDtypeStruct((NC * n_pad,), jnp.float32),
        ),
        mesh=mesh,
        scratch_types=scratch,
    )
    def body(center_hbm, j_hbm, s_hbm, eng_hbm, partial_hbm, **refs):
        idxs = [refs[f"idx{b}"] for b in range(3)]
        engs = [refs[f"eng{b}"] for b in range(3)]
        js = [refs[f"j{b}"] for b in range(2)]
        ss = [refs[f"s{b}"] for b in range(2)]
        acc_sh = refs["acc_sh"]
        in_sem = refs["in_sem"]
        scat_sem = refs["scat_sem"]
        wb_sem = refs["wb_sem"]

        cid = lax.axis_index("c")
        sid = lax.axis_index("s")
        wid = sid * NC + cid

        def fire_inputs(c, i3, i2):
            pltpu.async_copy(center_hbm.at[pl.ds(c * ROWS, ROWS)], idxs[i3],
                             in_sem)
            pltpu.async_copy(j_hbm.at[pl.ds(c * CHUNK, CHUNK)], js[i2],
                             in_sem)
            pltpu.async_copy(s_hbm.at[pl.ds(c * CHUNK, CHUNK)], ss[i2],
                             in_sem)

        def wait_inputs(c, i3, i2):
            pltpu.make_async_copy(center_hbm.at[pl.ds(c * ROWS, ROWS)],
                                  idxs[i3], in_sem).wait()
            pltpu.make_async_copy(j_hbm.at[pl.ds(c * CHUNK, CHUNK)], js[i2],
                                  in_sem).wait()
            pltpu.make_async_copy(s_hbm.at[pl.ds(c * CHUNK, CHUNK)], ss[i2],
                                  in_sem).wait()

        def drain_chunk(c, i3):
            idx_v, eng_v = idxs[i3], engs[i3]

            def drain_body(r, _):
                pltpu.make_async_copy(eng_v.at[pl.ds(r * ROW, ROW)],
                                      acc_sh.at[idx_v.at[r]],
                                      scat_sem).wait()
                return 0
            lax.fori_loop(0, ROWS, drain_body, 0)
            pltpu.make_async_copy(eng_v, eng_hbm.at[pl.ds(c * CHUNK, CHUNK)],
                                  wb_sem).wait()

        fire_inputs(wid, 0, 0)

        def zero_body(i, _):
            engs[0][pl.ds(i * LANES, LANES)] = jnp.zeros((LANES,),
                                                         jnp.float32)
            return 0
        lax.fori_loop(0, nps // LANES, zero_body, 0, unroll=8)
        pltpu.sync_copy(engs[0].at[pl.ds(0, nps)],
                        acc_sh.at[pl.ds(sid * nps, nps)])
        plsc.subcore_barrier()

        def process(k, i3, i2):
            c = wid + k * NW
            cp = wid + (k - 2) * NW
            cn = wid + (k + 1) * NW

            @pl.when(c < n_chunks)
            def _():
                wait_inputs(c, i3, i2)

            @pl.when((k >= 2) & (cp < n_chunks))
            def _():
                drain_chunk(cp, (i3 + 1) % 3)

            @pl.when(cn < n_chunks)
            def _():
                fire_inputs(cn, (i3 + 1) % 3, (i2 + 1) % 2)

            @pl.when(c < n_chunks)
            def _():
                idx_v, eng_v = idxs[i3], engs[i3]
                j_v, s_v = js[i2], ss[i2]

                def row_body(r, _):
                    for t in range(ROW // LANES):
                        sl = pl.ds(r * ROW + t * LANES, LANES)
                        eng_v[sl] = j_v[sl] * s_v[sl]
                    pltpu.async_copy(eng_v.at[pl.ds(r * ROW, ROW)],
                                     acc_sh.at[idx_v.at[r]], scat_sem,
                                     add=True)
                    return 0
                lax.fori_loop(0, ROWS, row_body, 0)

                pltpu.async_copy(eng_v, eng_hbm.at[pl.ds(c * CHUNK, CHUNK)],
                                 wb_sem)

        def rot_body(p, _):
            for jj in range(6):
                process(6 * p + jj, jj % 3, jj % 2)
            return 0
        lax.fori_loop(0, cpw // 6, rot_body, 0)

        for k in (cpw - 2, cpw - 1):
            c_last = wid + k * NW

            @pl.when(c_last < n_chunks)
            def _(c_last=c_last, k=k):
                drain_chunk(c_last, k % 3)

        plsc.subcore_barrier()
        pltpu.sync_copy(acc_sh.at[pl.ds(sid * nps, nps)],
                        engs[0].at[pl.ds(0, nps)])
        pltpu.sync_copy(engs[0].at[pl.ds(0, nps)],
                        partial_hbm.at[pl.ds(cid * n_pad + sid * nps, nps)])

    return body, n_pad


def _combine_kernel(p_ref, o_ref):
    o_ref[...] = (p_ref[0] + p_ref[1]) * FACTOR


def kernel(edge_index, atom_type, edge_J, edge_spin_distance):
    N = atom_type.shape[0]
    E = edge_J.shape[0]
    center2d = edge_index[0].reshape(E // ROW, ROW)
    j_flat = edge_J.reshape(E)

    sc_fn, n_pad = _sc_scatter_kernel(E, N)
    eng_flat, partial = sc_fn(center2d, j_flat, edge_spin_distance)

    p3 = partial.reshape(NC, n_pad // 128, 128)
    atom_pad = pl.pallas_call(
        _combine_kernel,
        out_shape=jax.ShapeDtypeStruct((n_pad // 128, 128), jnp.float32),
    )(p3)
    atom_eng = atom_pad.reshape(n_pad)[:N].reshape(N, 1)
    return eng_flat.reshape(E, 1), atom_eng
